# baseline (speedup 1.0000x reference)
import functools

import jax
import jax.numpy as jnp
from jax import lax
from jax.experimental import pallas as pl
from jax.experimental.pallas import tpu as pltpu
from jax.experimental.pallas import tpu_sc as plsc

N_ROWS = 100000
D = 128
NC = 2
NS = 16
NW = NC * NS

CHUNK = 128
NCH = 25
R = NCH * CHUNK
NBUF = 6

_mesh = plsc.VectorSubcoreMesh(core_axis_name="c", subcore_axis_name="s")

_scratch = (
    [pltpu.VMEM((R,), jnp.int32)]
    + [pltpu.VMEM((CHUNK, D), jnp.float32) for _ in range(NBUF)]
    + [pltpu.SemaphoreType.DMA for _ in range(2 * NBUF)]
)


@functools.partial(
    pl.kernel,
    out_type=jax.ShapeDtypeStruct((NW * R, D), jnp.float32),
    mesh=_mesh,
    scratch_types=_scratch,
)
def _gather_kernel(idx_hbm, tbl_hbm, out_hbm, idx_v, *rest):
    bufs = rest[:NBUF]
    gsems = rest[NBUF:2 * NBUF]
    ssems = rest[2 * NBUF:]

    w = lax.axis_index("s") * NC + lax.axis_index("c")
    r0 = w * R

    pltpu.sync_copy(idx_hbm.at[pl.ds(r0, R)], idx_v)

    def start_gather(k):
        b = k % NBUF
        return pltpu.async_copy(
            tbl_hbm.at[idx_v.at[pl.ds(k * CHUNK, CHUNK)]], bufs[b], gsems[b]
        )

    def start_store(k):
        b = k % NBUF
        return pltpu.async_copy(
            bufs[b], out_hbm.at[pl.ds(r0 + k * CHUNK, CHUNK)], ssems[b]
        )

    gather_cp = [None] * NCH
    store_cp = [None] * NCH
    for k in range(NBUF - 1):
        gather_cp[k] = start_gather(k)
    for k in range(NCH):
        gather_cp[k].wait()
        store_cp[k] = start_store(k)
        kn = k + NBUF - 1
        if kn < NCH:
            if k >= 1:
                store_cp[k - 1].wait()
            gather_cp[kn] = start_gather(kn)
    for k in range(NCH - NBUF, NCH):
        store_cp[k].wait()


def kernel(x, edge_index, batch, emb_table):
    idx = jnp.squeeze(x, axis=1)
    idx = jnp.pad(idx, (0, NW * R - N_ROWS))
    return _gather_kernel(idx, emb_table)[:N_ROWS]


# speedup vs baseline: 4.0254x; 4.0254x over previous
import functools

import jax
import jax.numpy as jnp
from jax import lax
from jax.experimental import pallas as pl
from jax.experimental.pallas import tpu as pltpu
from jax.experimental.pallas import tpu_sc as plsc

N_ROWS = 100000
D = 128
NC = 2
NS = 16
NW = NC * NS

CHUNK = 128
RW = 3128
NFULL = 23
TAIL_OFF = NFULL * CHUNK
TAIL_A = 128
TAIL_B = 56
TAIL_W31 = 88
IDX_PAD = (NW - 1) * RW + RW
NBUF = 6

_mesh = plsc.VectorSubcoreMesh(core_axis_name="c", subcore_axis_name="s")

_scratch = (
    [pltpu.VMEM((RW,), jnp.int32)]
    + [pltpu.VMEM((CHUNK, D), jnp.float32) for _ in range(NBUF)]
    + [pltpu.SemaphoreType.DMA for _ in range(2 * NBUF)]
)


@functools.partial(
    pl.kernel,
    out_type=jax.ShapeDtypeStruct((N_ROWS, D), jnp.float32),
    mesh=_mesh,
    scratch_types=_scratch,
)
def _gather_kernel(idx_hbm, tbl_hbm, out_hbm, idx_v, *rest):
    bufs = rest[:NBUF]
    gsems = rest[NBUF:2 * NBUF]
    ssems = rest[2 * NBUF:]

    w = lax.axis_index("s") * NC + lax.axis_index("c")
    r0 = w * RW

    pltpu.sync_copy(idx_hbm.at[pl.ds(r0, RW)], idx_v)

    def start_gather(k):
        b = k % NBUF
        return pltpu.async_copy(
            tbl_hbm.at[idx_v.at[pl.ds(k * CHUNK, CHUNK)]], bufs[b], gsems[b]
        )

    def start_store(k):
        b = k % NBUF
        return pltpu.async_copy(
            bufs[b], out_hbm.at[pl.ds(r0 + k * CHUNK, CHUNK)], ssems[b]
        )

    gather_cp = [None] * NFULL
    store_cp = [None] * NFULL
    for k in range(NBUF - 1):
        gather_cp[k] = start_gather(k)
    for k in range(NFULL):
        gather_cp[k].wait()
        store_cp[k] = start_store(k)
        kn = k + NBUF - 1
        if kn < NFULL:
            if k >= 1:
                store_cp[k - 1].wait()
            gather_cp[kn] = start_gather(kn)
    for k in range(max(0, NFULL - NBUF), NFULL):
        store_cp[k].wait()

    @pl.when(w < NW - 1)
    def _():
        g0 = pltpu.async_copy(
            tbl_hbm.at[idx_v.at[pl.ds(TAIL_OFF, TAIL_A)]], bufs[0], gsems[0]
        )
        g1 = pltpu.async_copy(
            tbl_hbm.at[idx_v.at[pl.ds(TAIL_OFF + TAIL_A, TAIL_B)]],
            bufs[1].at[pl.ds(0, TAIL_B)],
            gsems[1],
        )
        g0.wait()
        pltpu.sync_copy(bufs[0], out_hbm.at[pl.ds(r0 + TAIL_OFF, TAIL_A)])
        g1.wait()
        pltpu.sync_copy(
            bufs[1].at[pl.ds(0, TAIL_B)],
            out_hbm.at[pl.ds(r0 + TAIL_OFF + TAIL_A, TAIL_B)],
        )

    @pl.when(w == NW - 1)
    def _():
        g0 = pltpu.async_copy(
            tbl_hbm.at[idx_v.at[pl.ds(TAIL_OFF, TAIL_W31)]],
            bufs[0].at[pl.ds(0, TAIL_W31)],
            gsems[0],
        )
        g0.wait()
        pltpu.sync_copy(
            bufs[0].at[pl.ds(0, TAIL_W31)],
            out_hbm.at[pl.ds(r0 + TAIL_OFF, TAIL_W31)],
        )


def kernel(x, edge_index, batch, emb_table):
    idx = jnp.squeeze(x, axis=1)
    idx = jnp.pad(idx, (0, IDX_PAD - N_ROWS))
    return _gather_kernel(idx, emb_table)


# speedup vs baseline: 4.0390x; 1.0034x over previous
import functools

import jax
import jax.numpy as jnp
from jax import lax
from jax.experimental import pallas as pl
from jax.experimental.pallas import tpu as pltpu
from jax.experimental.pallas import tpu_sc as plsc

N_ROWS = 100000
D = 128
NC = 2
NS = 16
NW = NC * NS

CHUNK = 128
RW = 3128
NFULL = 23
TAIL_OFF = NFULL * CHUNK
TAIL_A = 128
TAIL_B = 56
TAIL_W31 = 88
RW31 = 3032
NBUF = 7

_mesh = plsc.VectorSubcoreMesh(core_axis_name="c", subcore_axis_name="s")

_scratch = (
    [pltpu.VMEM((RW,), jnp.int32)]
    + [pltpu.VMEM((CHUNK, D), jnp.float32) for _ in range(NBUF)]
    + [pltpu.SemaphoreType.DMA for _ in range(2 * NBUF)]
)


@functools.partial(
    pl.kernel,
    out_type=jax.ShapeDtypeStruct((N_ROWS, D), jnp.float32),
    mesh=_mesh,
    scratch_types=_scratch,
)
def _gather_kernel(idx_hbm, tbl_hbm, out_hbm, idx_v, *rest):
    bufs = rest[:NBUF]
    gsems = rest[NBUF:2 * NBUF]
    ssems = rest[2 * NBUF:]

    w = lax.axis_index("s") * NC + lax.axis_index("c")
    r0 = w * RW

    @pl.when(w < NW - 1)
    def _():
        pltpu.sync_copy(idx_hbm.at[pl.ds(r0, RW)], idx_v)

    @pl.when(w == NW - 1)
    def _():
        pltpu.sync_copy(idx_hbm.at[pl.ds(r0, RW31)], idx_v.at[pl.ds(0, RW31)])

    def start_gather(k):
        b = k % NBUF
        return pltpu.async_copy(
            tbl_hbm.at[idx_v.at[pl.ds(k * CHUNK, CHUNK)]], bufs[b], gsems[b]
        )

    def start_store(k):
        b = k % NBUF
        return pltpu.async_copy(
            bufs[b], out_hbm.at[pl.ds(r0 + k * CHUNK, CHUNK)], ssems[b]
        )

    gather_cp = [None] * NFULL
    store_cp = [None] * NFULL
    for k in range(NBUF - 1):
        gather_cp[k] = start_gather(k)
    for k in range(NFULL):
        gather_cp[k].wait()
        store_cp[k] = start_store(k)
        kn = k + NBUF - 1
        if kn < NFULL:
            if k >= 1:
                store_cp[k - 1].wait()
            gather_cp[kn] = start_gather(kn)
    for k in range(max(0, NFULL - NBUF), NFULL):
        store_cp[k].wait()

    @pl.when(w < NW - 1)
    def _():
        g0 = pltpu.async_copy(
            tbl_hbm.at[idx_v.at[pl.ds(TAIL_OFF, TAIL_A)]], bufs[0], gsems[0]
        )
        g1 = pltpu.async_copy(
            tbl_hbm.at[idx_v.at[pl.ds(TAIL_OFF + TAIL_A, TAIL_B)]],
            bufs[1].at[pl.ds(0, TAIL_B)],
            gsems[1],
        )
        g0.wait()
        pltpu.sync_copy(bufs[0], out_hbm.at[pl.ds(r0 + TAIL_OFF, TAIL_A)])
        g1.wait()
        pltpu.sync_copy(
            bufs[1].at[pl.ds(0, TAIL_B)],
            out_hbm.at[pl.ds(r0 + TAIL_OFF + TAIL_A, TAIL_B)],
        )

    @pl.when(w == NW - 1)
    def _():
        g0 = pltpu.async_copy(
            tbl_hbm.at[idx_v.at[pl.ds(TAIL_OFF, TAIL_W31)]],
            bufs[0].at[pl.ds(0, TAIL_W31)],
            gsems[0],
        )
        g0.wait()
        pltpu.sync_copy(
            bufs[0].at[pl.ds(0, TAIL_W31)],
            out_hbm.at[pl.ds(r0 + TAIL_OFF, TAIL_W31)],
        )


def kernel(x, edge_index, batch, emb_table):
    idx = jnp.squeeze(x, axis=1)
    return _gather_kernel(idx, emb_table)


# speedup vs baseline: 4.1158x; 1.0190x over previous
import functools

import jax
import jax.numpy as jnp
from jax import lax
from jax.experimental import pallas as pl
from jax.experimental.pallas import tpu as pltpu
from jax.experimental.pallas import tpu_sc as plsc

N_ROWS = 100000
D = 128
NC = 2
NS = 16
NW = NC * NS

CHUNK = 128
RW = 3128
NFULL = 23
TAIL_OFF = NFULL * CHUNK
TAIL_A = 128
TAIL_B = 56
TAIL_W31 = 88
RW31 = 3032
NBUF = 6

_mesh = plsc.VectorSubcoreMesh(core_axis_name="c", subcore_axis_name="s")

_scratch = (
    [pltpu.VMEM((RW,), jnp.int32)]
    + [pltpu.VMEM((CHUNK, D), jnp.float32) for _ in range(NBUF)]
    + [pltpu.SemaphoreType.DMA for _ in range(2 * NBUF)]
)


@functools.partial(
    pl.kernel,
    out_type=jax.ShapeDtypeStruct((N_ROWS, D), jnp.float32),
    mesh=_mesh,
    scratch_types=_scratch,
)
def _gather_kernel(idx_hbm, tbl_hbm, out_hbm, idx_v, *rest):
    bufs = rest[:NBUF]
    gsems = rest[NBUF:2 * NBUF]
    ssems = rest[2 * NBUF:]

    w = lax.axis_index("s") * NC + lax.axis_index("c")
    r0 = w * RW

    @pl.when(w < NW - 1)
    def _():
        pltpu.sync_copy(idx_hbm.at[pl.ds(r0, RW)], idx_v)

    @pl.when(w == NW - 1)
    def _():
        pltpu.sync_copy(idx_hbm.at[pl.ds(r0, RW31)], idx_v.at[pl.ds(0, RW31)])

    def start_gather(c, b):
        return pltpu.async_copy(
            tbl_hbm.at[idx_v.at[pl.ds(c * CHUNK, CHUNK)]], bufs[b], gsems[b]
        )

    def start_store(c, b):
        return pltpu.async_copy(
            bufs[b], out_hbm.at[pl.ds(r0 + c * CHUNK, CHUNK)], ssems[b]
        )

    def wait_gather(b):
        pltpu.make_async_copy(
            tbl_hbm.at[idx_v.at[pl.ds(0, CHUNK)]], bufs[b], gsems[b]
        ).wait()

    def wait_store(b):
        pltpu.make_async_copy(
            bufs[b], out_hbm.at[pl.ds(0, CHUNK)], ssems[b]
        ).wait()

    for c in range(NBUF - 1):
        start_gather(c, c)
    for c in range(NBUF):
        wait_gather(c % NBUF)
        start_store(c, c % NBUF)
        if c >= 1:
            wait_store((c - 1) % NBUF)
        start_gather(c + NBUF - 1, (c + NBUF - 1) % NBUF)

    @pl.loop(1, (NFULL - NBUF + 1) // NBUF)
    def _(i):
        c0 = i * NBUF
        for b in range(NBUF):
            wait_gather(b)
            start_store(c0 + b, b)
            wait_store((b + NBUF - 1) % NBUF)
            start_gather(c0 + b + NBUF - 1, (b + NBUF - 1) % NBUF)

    for c in range(NFULL - NBUF + 1, NFULL):
        wait_gather(c % NBUF)
        start_store(c, c % NBUF)
    for c in range(NFULL - NBUF, NFULL):
        wait_store(c % NBUF)

    @pl.when(w < NW - 1)
    def _():
        g0 = pltpu.async_copy(
            tbl_hbm.at[idx_v.at[pl.ds(TAIL_OFF, TAIL_A)]], bufs[0], gsems[0]
        )
        g1 = pltpu.async_copy(
            tbl_hbm.at[idx_v.at[pl.ds(TAIL_OFF + TAIL_A, TAIL_B)]],
            bufs[1].at[pl.ds(0, TAIL_B)],
            gsems[1],
        )
        g0.wait()
        pltpu.sync_copy(bufs[0], out_hbm.at[pl.ds(r0 + TAIL_OFF, TAIL_A)])
        g1.wait()
        pltpu.sync_copy(
            bufs[1].at[pl.ds(0, TAIL_B)],
            out_hbm.at[pl.ds(r0 + TAIL_OFF + TAIL_A, TAIL_B)],
        )

    @pl.when(w == NW - 1)
    def _():
        g0 = pltpu.async_copy(
            tbl_hbm.at[idx_v.at[pl.ds(TAIL_OFF, TAIL_W31)]],
            bufs[0].at[pl.ds(0, TAIL_W31)],
            gsems[0],
        )
        g0.wait()
        pltpu.sync_copy(
            bufs[0].at[pl.ds(0, TAIL_W31)],
            out_hbm.at[pl.ds(r0 + TAIL_OFF, TAIL_W31)],
        )


def kernel(x, edge_index, batch, emb_table):
    idx = jnp.squeeze(x, axis=1)
    return _gather_kernel(idx, emb_table)
